# Initial kernel scaffold; baseline (speedup 1.0000x reference)
import jax
import jax.numpy as jnp
from jax.experimental import pallas as pl


def kernel(query_points, ref_points, ref_flow, k):
    raise NotImplementedError("write your pallas kernel here")


# speedup vs baseline: 7.6017x; 7.6017x over previous
import jax
import jax.numpy as jnp
from jax.experimental import pallas as pl

_BN = 256


def _knn_body(q_ref, rt_ref, ft_ref, o_ref):
    q = q_ref[...]
    rt = rt_ref[...]
    bn = q.shape[0]
    m = rt.shape[1]
    q2 = jnp.sum(q * q, axis=1, keepdims=True)
    r2 = jnp.sum(rt * rt, axis=0, keepdims=True)
    qr = jax.lax.dot_general(q, rt, (((1,), (0,)), ((), ())),
                             preferred_element_type=jnp.float32)
    d2 = q2 + r2 - 2.0 * qr
    dist = jnp.sqrt(jnp.maximum(d2, 1e-12))
    iota = jax.lax.broadcasted_iota(jnp.int32, (bn, m), 1)
    wacc = jnp.zeros((bn, m), jnp.float32)
    wsum = jnp.zeros((bn, 1), jnp.float32)
    cur = dist
    for _ in range(3):
        dmin = jnp.min(cur, axis=1, keepdims=True)
        hit = cur == dmin
        idx = jnp.min(jnp.where(hit, iota, m), axis=1, keepdims=True)
        sel = iota == idx
        w = 1.0 / (dmin + 1e-8)
        wacc = jnp.where(sel, w, wacc)
        wsum = wsum + w
        cur = jnp.where(sel, jnp.float32(jnp.inf), cur)
    wacc = wacc * (1.0 / wsum)
    o_ref[...] = jax.lax.dot_general(wacc, ft_ref[...], (((1,), (1,)), ((), ())),
                                     preferred_element_type=jnp.float32)


def kernel(query_points, ref_points, ref_flow, k):
    del k
    n, d = query_points.shape
    m = ref_points.shape[0]
    rt = ref_points.T
    ft = ref_flow.T
    return pl.pallas_call(
        _knn_body,
        grid=(n // _BN,),
        in_specs=[
            pl.BlockSpec((_BN, d), lambda i: (i, 0)),
            pl.BlockSpec((d, m), lambda i: (0, 0)),
            pl.BlockSpec((d, m), lambda i: (0, 0)),
        ],
        out_specs=pl.BlockSpec((_BN, d), lambda i: (i, 0)),
        out_shape=jax.ShapeDtypeStruct((n, d), jnp.float32),
    )(query_points, rt, ft)


# speedup vs baseline: 8.1613x; 1.0736x over previous
import jax
import jax.numpy as jnp
from jax.experimental import pallas as pl

_BN = 256


def _knn_body(q_ref, rt_ref, ft_ref, o_ref):
    q = q_ref[...]
    rt = rt_ref[...]
    bn = q.shape[0]
    m = rt.shape[1]
    q2 = jnp.sum(q * q, axis=1, keepdims=True)
    r2 = jnp.sum(rt * rt, axis=0, keepdims=True)
    qr = jax.lax.dot_general(q, rt, (((1,), (0,)), ((), ())),
                             preferred_element_type=jnp.float32)
    d2 = q2 + r2 - 2.0 * qr
    iota = jax.lax.broadcasted_iota(jnp.int32, (bn, m), 1)
    wacc = jnp.zeros((bn, m), jnp.float32)
    wsum = jnp.zeros((bn, 1), jnp.float32)
    cur = d2
    for _ in range(3):
        dmin2 = jnp.min(cur, axis=1, keepdims=True)
        hit = cur == dmin2
        idx = jnp.min(jnp.where(hit, iota, m), axis=1, keepdims=True)
        sel = iota == idx
        dmin = jnp.sqrt(jnp.maximum(dmin2, 1e-12))
        w = 1.0 / (dmin + 1e-8)
        wacc = jnp.where(sel, w, wacc)
        wsum = wsum + w
        cur = jnp.where(sel, jnp.float32(jnp.inf), cur)
    wacc = wacc * (1.0 / wsum)
    o_ref[...] = jax.lax.dot_general(wacc, ft_ref[...], (((1,), (1,)), ((), ())),
                                     preferred_element_type=jnp.float32)


def kernel(query_points, ref_points, ref_flow, k):
    del k
    n, d = query_points.shape
    m = ref_points.shape[0]
    rt = ref_points.T
    ft = ref_flow.T
    return pl.pallas_call(
        _knn_body,
        grid=(n // _BN,),
        in_specs=[
            pl.BlockSpec((_BN, d), lambda i: (i, 0)),
            pl.BlockSpec((d, m), lambda i: (0, 0)),
            pl.BlockSpec((d, m), lambda i: (0, 0)),
        ],
        out_specs=pl.BlockSpec((_BN, d), lambda i: (i, 0)),
        out_shape=jax.ShapeDtypeStruct((n, d), jnp.float32),
    )(query_points, rt, ft)


# speedup vs baseline: 8.3908x; 1.0281x over previous
import functools

import jax
import jax.numpy as jnp
from jax import lax
from jax.experimental import pallas as pl
from jax.experimental.pallas import tpu as pltpu
from jax.experimental.pallas import tpu_sc as plsc

_BN = 256
_NW = 32
_L = 16


def _top3_body(q_ref, rt_ref, idx_ref, dst_ref):
    q = q_ref[...]
    rt = rt_ref[...]
    bn = q.shape[0]
    m = rt.shape[1]
    q2 = jnp.sum(q * q, axis=1, keepdims=True)
    r2 = jnp.sum(rt * rt, axis=0, keepdims=True)
    qr = jax.lax.dot_general(q, rt, (((1,), (0,)), ((), ())),
                             preferred_element_type=jnp.float32)
    d2 = q2 + r2 - 2.0 * qr
    dist = jnp.sqrt(jnp.maximum(d2, 1e-12))
    iota = jax.lax.broadcasted_iota(jnp.int32, (bn, m), 1)
    cur = dist
    idx_cols = []
    dst_cols = []
    for _ in range(3):
        dmin = jnp.min(cur, axis=1, keepdims=True)
        hit = cur == dmin
        idx = jnp.min(jnp.where(hit, iota, m), axis=1, keepdims=True)
        sel = iota == idx
        idx_cols.append(idx)
        dst_cols.append(dmin)
        cur = jnp.where(sel, jnp.float32(jnp.inf), cur)
    idx_ref[...] = jnp.concatenate(idx_cols, axis=1)
    dst_ref[...] = jnp.concatenate(dst_cols, axis=1)


def _tc_top3(query_points, rt):
    n, d = query_points.shape
    m = rt.shape[1]
    return pl.pallas_call(
        _top3_body,
        grid=(n // _BN,),
        in_specs=[
            pl.BlockSpec((_BN, d), lambda i: (i, 0)),
            pl.BlockSpec((d, m), lambda i: (0, 0)),
        ],
        out_specs=[
            pl.BlockSpec((_BN, 3), lambda i: (i, 0)),
            pl.BlockSpec((_BN, 3), lambda i: (i, 0)),
        ],
        out_shape=[
            jax.ShapeDtypeStruct((n, 3), jnp.int32),
            jax.ShapeDtypeStruct((n, 3), jnp.float32),
        ],
    )(query_points, rt)


def _sc_combine(flow_flat, idxs, dsts):
    n = idxs[0].shape[0]
    per_w = n // _NW
    nc = 2

    @functools.partial(
        pl.kernel,
        mesh=plsc.VectorSubcoreMesh(core_axis_name="c", subcore_axis_name="s"),
        compiler_params=pltpu.CompilerParams(needs_layout_passes=False),
        out_type=[jax.ShapeDtypeStruct((n,), jnp.float32)] * 3,
        scratch_types=(
            [pltpu.VMEM((flow_flat.shape[0],), jnp.float32)]
            + [pltpu.VMEM((per_w,), jnp.int32)] * 3
            + [pltpu.VMEM((per_w,), jnp.float32)] * 6
        ),
    )
    def sc_kernel(flow_hbm, i0_h, i1_h, i2_h, d0_h, d1_h, d2_h,
                  o0_h, o1_h, o2_h,
                  table_v, i0_v, i1_v, i2_v, d0_v, d1_v, d2_v,
                  o0_v, o1_v, o2_v):
        wid = lax.axis_index("s") * nc + lax.axis_index("c")
        base = wid * per_w
        iv_refs = (i0_v, i1_v, i2_v)
        dv_refs = (d0_v, d1_v, d2_v)
        ov_refs = (o0_v, o1_v, o2_v)
        pltpu.sync_copy(flow_hbm, table_v)
        for h, v in zip((i0_h, i1_h, i2_h, d0_h, d1_h, d2_h), iv_refs + dv_refs):
            pltpu.sync_copy(h.at[pl.ds(base, per_w)], v)
        for t in range(per_w // _L):
            s = t * _L
            iv = [r[pl.ds(s, _L)] for r in iv_refs]
            dv = [r[pl.ds(s, _L)] for r in dv_refs]
            w = [1.0 / (d + 1e-8) for d in dv]
            inv = 1.0 / (w[0] + w[1] + w[2])
            for c in range(3):
                acc = jnp.zeros((_L,), jnp.float32)
                for j in range(3):
                    g = plsc.load_gather(table_v, [iv[j] * 3 + c])
                    acc = acc + w[j] * g
                ov_refs[c][pl.ds(s, _L)] = acc * inv
        for v, h in zip(ov_refs, (o0_h, o1_h, o2_h)):
            pltpu.sync_copy(v, h.at[pl.ds(base, per_w)])

    return sc_kernel(flow_flat, *idxs, *dsts)


def kernel(query_points, ref_points, ref_flow, k):
    del k
    knn_idx, knn_dst = _tc_top3(query_points, ref_points.T)
    o0, o1, o2 = _sc_combine(
        ref_flow.reshape(-1),
        tuple(knn_idx[:, j] for j in range(3)),
        tuple(knn_dst[:, j] for j in range(3)),
    )
    return jnp.stack([o0, o1, o2], axis=1)


# speedup vs baseline: 9.3377x; 1.1128x over previous
import functools

import jax
import jax.numpy as jnp
from jax import lax
from jax.experimental import pallas as pl
from jax.experimental.pallas import tpu as pltpu
from jax.experimental.pallas import tpu_sc as plsc

_BN = 256
_NW = 32
_L = 16


def _top3_body(q_ref, rt_ref, idx_ref, dst_ref):
    q = q_ref[...]
    rt = rt_ref[...]
    bn = q.shape[0]
    m = rt.shape[1]
    q2 = jnp.sum(q * q, axis=1, keepdims=True)
    r2 = jnp.sum(rt * rt, axis=0, keepdims=True)
    qr = jax.lax.dot_general(q, rt, (((1,), (0,)), ((), ())),
                             preferred_element_type=jnp.float32)
    d2 = q2 + r2 - 2.0 * qr
    dist = jnp.sqrt(jnp.maximum(d2, 1e-12))
    iota = jax.lax.broadcasted_iota(jnp.int32, (bn, m), 1).astype(jnp.float32)
    m_f = jnp.float32(m)
    cur = dist
    idx_cols = []
    dst_cols = []
    for r in range(3):
        dmin = jnp.min(cur, axis=1, keepdims=True)
        hit = cur == dmin
        idx = jnp.min(jnp.where(hit, iota, m_f), axis=1, keepdims=True)
        idx_cols.append(idx)
        dst_cols.append(dmin)
        if r < 2:
            sel = iota == idx
            cur = jnp.where(sel, jnp.float32(jnp.inf), cur)
    idx_ref[...] = jnp.concatenate(idx_cols, axis=1).astype(jnp.int32)
    dst_ref[...] = jnp.concatenate(dst_cols, axis=1)


def _tc_top3(query_points, rt):
    n, d = query_points.shape
    m = rt.shape[1]
    return pl.pallas_call(
        _top3_body,
        grid=(n // _BN,),
        in_specs=[
            pl.BlockSpec((_BN, d), lambda i: (i, 0)),
            pl.BlockSpec((d, m), lambda i: (0, 0)),
        ],
        out_specs=[
            pl.BlockSpec((_BN, 3), lambda i: (i, 0)),
            pl.BlockSpec((_BN, 3), lambda i: (i, 0)),
        ],
        out_shape=[
            jax.ShapeDtypeStruct((n, 3), jnp.int32),
            jax.ShapeDtypeStruct((n, 3), jnp.float32),
        ],
    )(query_points, rt)


def _sc_combine(flow_flat, idxs, dsts):
    n = idxs[0].shape[0]
    per_w = n // _NW
    nc = 2

    @functools.partial(
        pl.kernel,
        mesh=plsc.VectorSubcoreMesh(core_axis_name="c", subcore_axis_name="s"),
        compiler_params=pltpu.CompilerParams(needs_layout_passes=False),
        out_type=[jax.ShapeDtypeStruct((n,), jnp.float32)] * 3,
        scratch_types=(
            [pltpu.VMEM((flow_flat.shape[0],), jnp.float32)]
            + [pltpu.VMEM((per_w,), jnp.int32)] * 3
            + [pltpu.VMEM((per_w,), jnp.float32)] * 6
        ),
    )
    def sc_kernel(flow_hbm, i0_h, i1_h, i2_h, d0_h, d1_h, d2_h,
                  o0_h, o1_h, o2_h,
                  table_v, i0_v, i1_v, i2_v, d0_v, d1_v, d2_v,
                  o0_v, o1_v, o2_v):
        wid = lax.axis_index("s") * nc + lax.axis_index("c")
        base = wid * per_w
        iv_refs = (i0_v, i1_v, i2_v)
        dv_refs = (d0_v, d1_v, d2_v)
        ov_refs = (o0_v, o1_v, o2_v)
        pltpu.sync_copy(flow_hbm, table_v)
        for h, v in zip((i0_h, i1_h, i2_h, d0_h, d1_h, d2_h), iv_refs + dv_refs):
            pltpu.sync_copy(h.at[pl.ds(base, per_w)], v)
        for t in range(per_w // _L):
            s = t * _L
            iv = [r[pl.ds(s, _L)] for r in iv_refs]
            dv = [r[pl.ds(s, _L)] for r in dv_refs]
            w = [1.0 / (d + 1e-8) for d in dv]
            inv = 1.0 / (w[0] + w[1] + w[2])
            for c in range(3):
                acc = jnp.zeros((_L,), jnp.float32)
                for j in range(3):
                    g = plsc.load_gather(table_v, [iv[j] * 3 + c])
                    acc = acc + w[j] * g
                ov_refs[c][pl.ds(s, _L)] = acc * inv
        for v, h in zip(ov_refs, (o0_h, o1_h, o2_h)):
            pltpu.sync_copy(v, h.at[pl.ds(base, per_w)])

    return sc_kernel(flow_flat, *idxs, *dsts)


def kernel(query_points, ref_points, ref_flow, k):
    del k
    knn_idx, knn_dst = _tc_top3(query_points, ref_points.T)
    o0, o1, o2 = _sc_combine(
        ref_flow.reshape(-1),
        tuple(knn_idx[:, j] for j in range(3)),
        tuple(knn_dst[:, j] for j in range(3)),
    )
    return jnp.stack([o0, o1, o2], axis=1)


# speedup vs baseline: 9.7909x; 1.0485x over previous
import functools

import jax
import jax.numpy as jnp
from jax import lax
from jax.experimental import pallas as pl
from jax.experimental.pallas import tpu as pltpu
from jax.experimental.pallas import tpu_sc as plsc

_BN = 256
_NW = 32
_L = 16


def _top3_body(q_ref, rt_ref, idx_ref, dst_ref):
    q4 = q_ref[...]
    r4 = rt_ref[...]
    bn = q4.shape[0]
    m = r4.shape[1]
    q2 = jnp.sum(q4 * q4, axis=1, keepdims=True) - 1.0
    qr = jax.lax.dot_general(q4, r4, (((1,), (0,)), ((), ())),
                             preferred_element_type=jnp.float32)
    d2 = qr + q2
    dist = jnp.sqrt(jnp.maximum(d2, 1e-12))
    iota = jax.lax.broadcasted_iota(jnp.int32, (bn, m), 1).astype(jnp.float32)
    m_f = jnp.float32(m)
    cur = dist
    idx_cols = []
    dst_cols = []
    for r in range(3):
        dmin = jnp.min(cur, axis=1, keepdims=True)
        hit = cur == dmin
        idx = jnp.min(jnp.where(hit, iota, m_f), axis=1, keepdims=True)
        idx_cols.append(idx)
        dst_cols.append(dmin)
        if r < 2:
            sel = iota == idx
            cur = jnp.where(sel, jnp.float32(jnp.inf), cur)
    idx_ref[...] = jnp.concatenate(idx_cols, axis=1).astype(jnp.int32)
    dst_ref[...] = jnp.concatenate(dst_cols, axis=1)


def _tc_top3(query_points, ref_points):
    n = query_points.shape[0]
    m = ref_points.shape[0]
    q4 = jnp.concatenate(
        [query_points, jnp.ones((n, 1), jnp.float32)], axis=1)
    r4 = jnp.concatenate(
        [ref_points.T * -2.0,
         jnp.sum(ref_points * ref_points, axis=1)[None, :]], axis=0)
    return pl.pallas_call(
        _top3_body,
        grid=(n // _BN,),
        in_specs=[
            pl.BlockSpec((_BN, 4), lambda i: (i, 0)),
            pl.BlockSpec((4, m), lambda i: (0, 0)),
        ],
        out_specs=[
            pl.BlockSpec((_BN, 3), lambda i: (i, 0)),
            pl.BlockSpec((_BN, 3), lambda i: (i, 0)),
        ],
        out_shape=[
            jax.ShapeDtypeStruct((n, 3), jnp.int32),
            jax.ShapeDtypeStruct((n, 3), jnp.float32),
        ],
    )(q4, r4)


def _sc_combine(flow_flat, idxs, dsts):
    n = idxs[0].shape[0]
    per_w = n // _NW
    nc = 2

    @functools.partial(
        pl.kernel,
        mesh=plsc.VectorSubcoreMesh(core_axis_name="c", subcore_axis_name="s"),
        compiler_params=pltpu.CompilerParams(needs_layout_passes=False),
        out_type=[jax.ShapeDtypeStruct((n,), jnp.float32)] * 3,
        scratch_types=(
            [pltpu.VMEM((flow_flat.shape[0],), jnp.float32)]
            + [pltpu.VMEM((per_w,), jnp.int32)] * 3
            + [pltpu.VMEM((per_w,), jnp.float32)] * 6
        ),
    )
    def sc_kernel(flow_hbm, i0_h, i1_h, i2_h, d0_h, d1_h, d2_h,
                  o0_h, o1_h, o2_h,
                  table_v, i0_v, i1_v, i2_v, d0_v, d1_v, d2_v,
                  o0_v, o1_v, o2_v):
        wid = lax.axis_index("s") * nc + lax.axis_index("c")
        base = wid * per_w
        iv_refs = (i0_v, i1_v, i2_v)
        dv_refs = (d0_v, d1_v, d2_v)
        ov_refs = (o0_v, o1_v, o2_v)
        pltpu.sync_copy(flow_hbm, table_v)
        for h, v in zip((i0_h, i1_h, i2_h, d0_h, d1_h, d2_h), iv_refs + dv_refs):
            pltpu.sync_copy(h.at[pl.ds(base, per_w)], v)
        for t in range(per_w // _L):
            s = t * _L
            iv = [r[pl.ds(s, _L)] for r in iv_refs]
            dv = [r[pl.ds(s, _L)] for r in dv_refs]
            w = [1.0 / (d + 1e-8) for d in dv]
            inv = 1.0 / (w[0] + w[1] + w[2])
            for c in range(3):
                acc = jnp.zeros((_L,), jnp.float32)
                for j in range(3):
                    g = plsc.load_gather(table_v, [iv[j] * 3 + c])
                    acc = acc + w[j] * g
                ov_refs[c][pl.ds(s, _L)] = acc * inv
        for v, h in zip(ov_refs, (o0_h, o1_h, o2_h)):
            pltpu.sync_copy(v, h.at[pl.ds(base, per_w)])

    return sc_kernel(flow_flat, *idxs, *dsts)


def kernel(query_points, ref_points, ref_flow, k):
    del k
    knn_idx, knn_dst = _tc_top3(query_points, ref_points)
    o0, o1, o2 = _sc_combine(
        ref_flow.reshape(-1),
        tuple(knn_idx[:, j] for j in range(3)),
        tuple(knn_dst[:, j] for j in range(3)),
    )
    return jnp.stack([o0, o1, o2], axis=1)
